# Initial kernel scaffold; baseline (speedup 1.0000x reference)
import jax
import jax.numpy as jnp
from jax.experimental import pallas as pl


def kernel(x, edge_index, edge_value, params):
    raise NotImplementedError("write your pallas kernel here")


# speedup vs baseline: 2.6553x; 2.6553x over previous
import functools

import jax
import jax.numpy as jnp
from jax import lax
from jax.experimental import pallas as pl
from jax.experimental.pallas import tpu as pltpu
from jax.experimental.pallas import tpu_sc as plsc

N_OBS = 10000
F = 64
E = 320000
NUM_LAYERS = 3

E_BLK = 5000
NB = E // E_BLK

NC = 2
NS = 16
E_PER_SC = E // NC
E_PER_W = E_PER_SC // NS
CH = 80
NCH = E_PER_W // CH
RPAD = 640
NPAD = NS * RPAD

_f32 = jnp.float32


def _k1_body(layer0, ee_ref, dstf_ref, psrc_ref, feat_ref, wmf_f_ref,
             wcat_ref, bias_ref, mof_ref, epe_ref, aggf_ref, cnt_ref):
    ee = ee_ref[...]
    if layer0:
        ep = ee * wcat_ref[...]
    else:
        ep = jnp.dot(ee, wcat_ref[...], preferred_element_type=_f32)
    iota = lax.broadcasted_iota(jnp.int32, (1, F), 1).astype(_f32)
    oh = (dstf_ref[...] == iota).astype(_f32)
    t = jnp.dot(feat_ref[...], wmf_f_ref[...], preferred_element_type=_f32)
    bmf = bias_ref[:, :F]
    bmo = bias_ref[:, F:2 * F]
    m_of = jnp.maximum(jnp.dot(oh, t, preferred_element_type=_f32)
                       + ep[:, :F] + bmf, 0.0)
    mof_ref[...] = m_of
    m_fo = jnp.maximum(psrc_ref[...] + ep[:, F:2 * F] + bmo, 0.0)
    epe_ref[...] = ep[:, 2 * F:]

    @pl.when(pl.program_id(0) == 0)
    def _():
        aggf_ref[...] = jnp.zeros_like(aggf_ref)
        cnt_ref[...] = jnp.zeros_like(cnt_ref)

    aggf_ref[...] += lax.dot_general(oh, m_fo, (((0,), (0,)), ((), ())),
                                     preferred_element_type=_f32)
    ones_col = jnp.ones((E_BLK, 1), _f32)
    cnt_ref[...] += lax.dot_general(oh, ones_col, (((0,), (0,)), ((), ())),
                                    preferred_element_type=_f32)


def _run_k1(layer0, edge, dstf, psrc, feat, wmf_f, wcat, bias):
    ein = edge.shape[1]
    psrc_spec = (pl.BlockSpec((1, F), lambda i: (0, 0)) if layer0
                 else pl.BlockSpec((E_BLK, F), lambda i: (i, 0)))
    return pl.pallas_call(
        functools.partial(_k1_body, layer0),
        grid=(NB,),
        in_specs=[
            pl.BlockSpec((E_BLK, ein), lambda i: (i, 0)),
            pl.BlockSpec((E_BLK, 1), lambda i: (i, 0)),
            psrc_spec,
            pl.BlockSpec((F, F), lambda i: (0, 0)),
            pl.BlockSpec((F, F), lambda i: (0, 0)),
            pl.BlockSpec((ein, 3 * F), lambda i: (0, 0)),
            pl.BlockSpec((1, 3 * F), lambda i: (0, 0)),
        ],
        out_specs=[
            pl.BlockSpec((E_BLK, F), lambda i: (i, 0)),
            pl.BlockSpec((E_BLK, F), lambda i: (i, 0)),
            pl.BlockSpec((F, F), lambda i: (0, 0)),
            pl.BlockSpec((F, 1), lambda i: (0, 0)),
        ],
        out_shape=[
            jax.ShapeDtypeStruct((E, F), _f32),
            jax.ShapeDtypeStruct((E, F), _f32),
            jax.ShapeDtypeStruct((F, F), _f32),
            jax.ShapeDtypeStruct((F, 1), _f32),
        ],
        compiler_params=pltpu.CompilerParams(
            dimension_semantics=("arbitrary",)),
    )(edge, dstf, psrc, feat, wmf_f, wcat, bias)


def _k3_body(last, node_ref, aggo_ref, cntp_ref, aggf_ref, cntd_ref,
             feat_ref, wn_n_ref, wn_a_ref, bn_ref, wf_f_ref, wf_a_ref,
             bf_ref, we_n_ref, we_f_ref, wmo_nx_ref,
             node2_ref, a_ref, pn_ref, feat2_ref, b_ref):
    node = node_ref[...]
    cnt = jnp.maximum(cntp_ref[0][:N_OBS, 0:1] + cntp_ref[1][:N_OBS, 0:1], 1.0)
    aggo = (aggo_ref[0][:N_OBS] + aggo_ref[1][:N_OBS]) / cnt
    new_node = jnp.maximum(
        jnp.dot(node, wn_n_ref[...], preferred_element_type=_f32)
        + jnp.dot(aggo, wn_a_ref[...], preferred_element_type=_f32)
        + bn_ref[...], 0.0)
    node2 = jnp.maximum(new_node + node, 0.0)
    node2_ref[...] = node2
    a_ref[...] = jnp.dot(new_node, we_n_ref[...], preferred_element_type=_f32)
    if last:
        pn_ref[...] = jnp.zeros_like(pn_ref)
    else:
        pn_ref[...] = jnp.dot(node2, wmo_nx_ref[...],
                              preferred_element_type=_f32)
    feat = feat_ref[...]
    aggf = aggf_ref[...] / jnp.maximum(cntd_ref[...], 1.0)
    new_feat = jnp.maximum(
        jnp.dot(feat, wf_f_ref[...], preferred_element_type=_f32)
        + jnp.dot(aggf, wf_a_ref[...], preferred_element_type=_f32)
        + bf_ref[...], 0.0)
    feat2 = jnp.maximum(new_feat + feat, 0.0)
    feat2_ref[...] = feat2
    b_ref[...] = jnp.dot(new_feat, we_f_ref[...], preferred_element_type=_f32)


def _run_k3(last, node, aggo_parts, cnt_parts, aggf, cntd, feat,
            wn_n, wn_a, bn, wf_f, wf_a, bf, we_n, we_f, wmo_nx):
    return pl.pallas_call(
        functools.partial(_k3_body, last),
        out_shape=[
            jax.ShapeDtypeStruct((N_OBS, F), _f32),
            jax.ShapeDtypeStruct((N_OBS, F), _f32),
            jax.ShapeDtypeStruct((N_OBS, F), _f32),
            jax.ShapeDtypeStruct((F, F), _f32),
            jax.ShapeDtypeStruct((F, F), _f32),
        ],
    )(node, aggo_parts, cnt_parts, aggf, cntd, feat,
      wn_n, wn_a, bn, wf_f, wf_a, bf, we_n, we_f, wmo_nx)


def _k5_body(epe_ref, asrc_ref, dstf_ref, prev_ref, b_ref, be_ref, out_ref):
    iota = lax.broadcasted_iota(jnp.int32, (1, F), 1).astype(_f32)
    oh = (dstf_ref[...] == iota).astype(_f32)
    new_edge = jnp.maximum(
        epe_ref[...] + asrc_ref[...]
        + jnp.dot(oh, b_ref[...], preferred_element_type=_f32)
        + be_ref[...], 0.0)
    out_ref[...] = jnp.maximum(new_edge + prev_ref[...], 0.0)


def _run_k5(epe, asrc, dstf, prev, b, be):
    ein = prev.shape[1]
    return pl.pallas_call(
        _k5_body,
        grid=(NB,),
        in_specs=[
            pl.BlockSpec((E_BLK, F), lambda i: (i, 0)),
            pl.BlockSpec((E_BLK, F), lambda i: (i, 0)),
            pl.BlockSpec((E_BLK, 1), lambda i: (i, 0)),
            pl.BlockSpec((E_BLK, ein), lambda i: (i, 0)),
            pl.BlockSpec((F, F), lambda i: (0, 0)),
            pl.BlockSpec((1, F), lambda i: (0, 0)),
        ],
        out_specs=pl.BlockSpec((E_BLK, F), lambda i: (i, 0)),
        out_shape=jax.ShapeDtypeStruct((E, F), _f32),
    )(epe, asrc, dstf, prev, b, be)


OB = 200


def _k6_body(node_ref, feat_ref, wo_ref, wfh_ref, bh_ref, wout_ref, bout_ref,
             w1_ref, b1_ref, w2_ref, b2_ref, dhat_ref, yhat_ref):
    obs_h = jnp.dot(node_ref[...], wo_ref[...], preferred_element_type=_f32)
    feat_h = jnp.dot(feat_ref[...], wfh_ref[...], preferred_element_type=_f32)
    h = jnp.maximum(obs_h[:, None, :] + feat_h[None, :, :]
                    + bh_ref[...][None, :, :], 0.0)
    dhat = jnp.sum(h * wout_ref[...][None, :, :], axis=2) + bout_ref[...]
    dhat_ref[...] = dhat
    hid = jnp.maximum(jnp.dot(dhat, w1_ref[...], preferred_element_type=_f32)
                      + b1_ref[...], 0.0)
    yhat_ref[...] = (jnp.dot(hid, w2_ref[...], preferred_element_type=_f32)
                     + b2_ref[...])


def _run_k6(node, feat, wo, wfh, bh, wout, bout, w1, b1, w2, b2):
    nblk = N_OBS // OB
    return pl.pallas_call(
        _k6_body,
        grid=(nblk,),
        in_specs=[
            pl.BlockSpec((OB, F), lambda i: (i, 0)),
            pl.BlockSpec((F, F), lambda i: (0, 0)),
            pl.BlockSpec((F, F), lambda i: (0, 0)),
            pl.BlockSpec((F, F), lambda i: (0, 0)),
            pl.BlockSpec((1, F), lambda i: (0, 0)),
            pl.BlockSpec((1, F), lambda i: (0, 0)),
            pl.BlockSpec((1, 1), lambda i: (0, 0)),
            pl.BlockSpec((F, F), lambda i: (0, 0)),
            pl.BlockSpec((1, F), lambda i: (0, 0)),
            pl.BlockSpec((F, 1), lambda i: (0, 0)),
            pl.BlockSpec((1, 1), lambda i: (0, 0)),
        ],
        out_specs=[
            pl.BlockSpec((OB, F), lambda i: (i, 0)),
            pl.BlockSpec((OB, 1), lambda i: (i, 0)),
        ],
        out_shape=[
            jax.ShapeDtypeStruct((N_OBS, F), _f32),
            jax.ShapeDtypeStruct((N_OBS, 1), _f32),
        ],
    )(node, feat, wo, wfh, bh, wout, bout, w1, b1, w2, b2)


@functools.lru_cache(maxsize=None)
def _sc_mesh():
    return plsc.VectorSubcoreMesh(core_axis_name="c", subcore_axis_name="s")


def _fill_vmem(ref, nrows, ncols, val):
    v = jnp.full((16,), val, _f32)

    def frow(r, _):
        def fcol(j, _):
            ref[r, pl.ds(j * 16, 16)] = v
            return 0
        return lax.fori_loop(0, ncols // 16, fcol, 0)
    lax.fori_loop(0, nrows, frow, 0)


def _scatter_body(with_count, mof_hbm, src_hbm, out_hbm, cnt_hbm,
                  rows_v, idx_v, zbuf_v, ones_v, cbuf_v, acc_sh, cacc_sh):
    cid = lax.axis_index("c")
    sid = lax.axis_index("s")
    _fill_vmem(zbuf_v, RPAD, F, 0.0)
    pltpu.sync_copy(zbuf_v, acc_sh.at[pl.ds(sid * RPAD, RPAD)])
    if with_count:
        _fill_vmem(ones_v, CH, 16, 1.0)
        _fill_vmem(cbuf_v, RPAD, 16, 0.0)
        pltpu.sync_copy(cbuf_v, cacc_sh.at[pl.ds(sid * RPAD, RPAD)])
    plsc.subcore_barrier()

    def chunk(c, _):
        b = cid * E_PER_SC + sid * E_PER_W + c * CH
        pltpu.sync_copy(src_hbm.at[pl.ds(b, CH)], idx_v)
        pltpu.sync_copy(mof_hbm.at[pl.ds(b, CH)], rows_v)
        pltpu.sync_copy(rows_v, acc_sh.at[idx_v], add=True)
        if with_count:
            pltpu.sync_copy(ones_v, cacc_sh.at[idx_v], add=True)
        return 0
    lax.fori_loop(0, NCH, chunk, 0)
    plsc.subcore_barrier()
    pltpu.sync_copy(acc_sh.at[pl.ds(sid * RPAD, RPAD)], zbuf_v)
    pltpu.sync_copy(zbuf_v, out_hbm.at[pl.ds(cid * NPAD + sid * RPAD, RPAD)])
    if with_count:
        pltpu.sync_copy(cacc_sh.at[pl.ds(sid * RPAD, RPAD)], cbuf_v)
        pltpu.sync_copy(cbuf_v, cnt_hbm.at[pl.ds(cid * NPAD + sid * RPAD, RPAD)])


@functools.lru_cache(maxsize=None)
def _make_scatter(with_count):
    return pl.kernel(
        functools.partial(_scatter_body, with_count),
        out_type=[
            jax.ShapeDtypeStruct((NC * NPAD, F), _f32),
            jax.ShapeDtypeStruct((NC * NPAD, 16), _f32),
        ],
        mesh=_sc_mesh(),
        compiler_params=pltpu.CompilerParams(use_tc_tiling_on_sc=False),
        scratch_types=[
            pltpu.VMEM((CH, F), _f32),
            pltpu.VMEM((CH,), jnp.int32),
            pltpu.VMEM((RPAD, F), _f32),
            pltpu.VMEM((CH, 16), _f32),
            pltpu.VMEM((RPAD, 16), _f32),
            pltpu.VMEM_SHARED((NPAD, F), _f32),
            pltpu.VMEM_SHARED((NPAD, 16), _f32),
        ],
    )


def _sc_scatter(mof, src, with_count):
    out, cnt = _make_scatter(with_count)(mof, src)
    return out.reshape(NC, NPAD, F), cnt.reshape(NC, NPAD, 16)


def _gather_body(ntab, *refs):
    tabs = refs[:ntab]
    src_hbm = refs[ntab]
    outs = refs[ntab + 1:2 * ntab + 1]
    idx_v = refs[2 * ntab + 1]
    buf_v = refs[2 * ntab + 2]
    sem = refs[2 * ntab + 3]
    cid = lax.axis_index("c")
    sid = lax.axis_index("s")

    def chunk(c, _):
        b = cid * E_PER_SC + sid * E_PER_W + c * CH
        pltpu.sync_copy(src_hbm.at[pl.ds(b, CH)], idx_v)
        for t in range(ntab):
            pltpu.async_copy(tabs[t].at[idx_v], buf_v, sem).wait()
            pltpu.sync_copy(buf_v, outs[t].at[pl.ds(b, CH)])
        return 0
    lax.fori_loop(0, NCH, chunk, 0)


@functools.lru_cache(maxsize=None)
def _make_gather(ntab):
    return pl.kernel(
        functools.partial(_gather_body, ntab),
        out_type=[jax.ShapeDtypeStruct((E, F), _f32) for _ in range(ntab)],
        mesh=_sc_mesh(),
        compiler_params=pltpu.CompilerParams(use_tc_tiling_on_sc=False),
        scratch_types=[
            pltpu.VMEM((CH,), jnp.int32),
            pltpu.VMEM((CH, F), _f32),
            pltpu.SemaphoreType.DMA,
        ],
    )


def _gather2(a, b, src):
    return _make_gather(2)(a, b, src)


def _gather1(a, src):
    return _make_gather(1)(a, src)


def kernel(x, edge_index, edge_value, params):
    src = edge_index[0].astype(jnp.int32)
    dstf = edge_index[1].astype(_f32).reshape(E, 1)
    n = N_OBS

    node_emb = jnp.ones((n, F), _f32)
    feature_emb = jnp.eye(F, dtype=_f32)
    edge_emb = edge_value.reshape(E, 1)

    def split(p, ein):
        wmf_f, wmf_e = p['Wmf'][:F], p['Wmf'][F:]
        wmo_n, wmo_e = p['Wmo'][:F], p['Wmo'][F:]
        we_e = p['We'][:ein]
        we_n = p['We'][ein:ein + F]
        we_f = p['We'][ein + F:]
        wcat = jnp.concatenate([wmf_e, wmo_e, we_e], axis=1)
        bias = jnp.concatenate([p['bmf'], p['bmo'], p['be']]).reshape(1, 3 * F)
        return wmf_f, wmo_n, wcat, bias, we_n, we_f

    blocks = [params['block%d' % i] for i in range(NUM_LAYERS)]
    sp = [split(blocks[i], 1 if i == 0 else F) for i in range(NUM_LAYERS)]

    psrc = None
    cnt_parts = None
    for i in range(NUM_LAYERS):
        p = blocks[i]
        wmf_f, wmo_n, wcat, bias, we_n, we_f = sp[i]
        layer0 = (i == 0)
        last = (i == NUM_LAYERS - 1)
        if layer0:
            psrc_in = jnp.sum(wmo_n, axis=0, keepdims=True)
        else:
            psrc_in = psrc
        mof, epe, aggf_sum, cntd = _run_k1(
            layer0, edge_emb, dstf, psrc_in, feature_emb, wmf_f, wcat, bias)
        aggo_parts, cparts = _sc_scatter(mof, src, with_count=layer0)
        if layer0:
            cnt_parts = cparts
        wmo_nx = sp[i + 1][1] if not last else jnp.zeros((F, F), _f32)
        node2, a_tab, pn_tab, feat2, b_tab = _run_k3(
            last, node_emb, aggo_parts, cnt_parts, aggf_sum, cntd,
            feature_emb,
            p['Wn'][:F], p['Wn'][F:], p['bn'].reshape(1, F),
            p['Wf'][:F], p['Wf'][F:], p['bf'].reshape(1, F),
            we_n, we_f, wmo_nx)
        if last:
            (asrc,) = _gather1(a_tab, src)
        else:
            asrc, psrc = _gather2(a_tab, pn_tab, src)
        be = bias[:, 2 * F:]
        edge_emb = _run_k5(epe, asrc, dstf, edge_emb, b_tab, be)
        node_emb = node2
        feature_emb = feat2

    ep = params['eph']
    npar = params['nph']
    d_hat, y_hat = _run_k6(
        node_emb, feature_emb, ep['Wo'], ep['Wf'], ep['bh'].reshape(1, F),
        ep['wout'].reshape(1, F), ep['bout'].reshape(1, 1),
        npar['W1'], npar['b1'].reshape(1, F),
        npar['W2'], npar['b2'].reshape(1, 1))
    return d_hat, y_hat


# speedup vs baseline: 4.6730x; 1.7599x over previous
import functools

import jax
import jax.numpy as jnp
from jax import lax
from jax.experimental import pallas as pl
from jax.experimental.pallas import tpu as pltpu
from jax.experimental.pallas import tpu_sc as plsc

N_OBS = 10000
F = 64
E = 320000
NUM_LAYERS = 3

E_BLK = 5000
NB = E // E_BLK

NC = 2
NS = 16
E_PER_SC = E // NC
E_PER_W = E_PER_SC // NS
CH = 80
NCH = E_PER_W // CH
RPAD = 640
NPAD = NS * RPAD

_f32 = jnp.float32


def _k1_body(layer0, ee_ref, dstf_ref, psrc_ref, feat_ref, wmf_f_ref,
             wcat_ref, bias_ref, mof_ref, epe_ref, aggf_ref, cnt_ref):
    ee = ee_ref[...]
    if layer0:
        ep = ee * wcat_ref[...]
    else:
        ep = jnp.dot(ee, wcat_ref[...], preferred_element_type=_f32)
    iota = lax.broadcasted_iota(jnp.int32, (1, F), 1).astype(_f32)
    oh = (dstf_ref[...] == iota).astype(_f32)
    t = jnp.dot(feat_ref[...], wmf_f_ref[...], preferred_element_type=_f32)
    bmf = bias_ref[:, :F]
    bmo = bias_ref[:, F:2 * F]
    m_of = jnp.maximum(jnp.dot(oh, t, preferred_element_type=_f32)
                       + ep[:, :F] + bmf, 0.0)
    mof_ref[...] = m_of
    m_fo = jnp.maximum(psrc_ref[...] + ep[:, F:2 * F] + bmo, 0.0)
    epe_ref[...] = ep[:, 2 * F:]

    @pl.when(pl.program_id(0) == 0)
    def _():
        aggf_ref[...] = jnp.zeros_like(aggf_ref)
        cnt_ref[...] = jnp.zeros_like(cnt_ref)

    aggf_ref[...] += lax.dot_general(oh, m_fo, (((0,), (0,)), ((), ())),
                                     preferred_element_type=_f32)
    ones_col = jnp.ones((E_BLK, 1), _f32)
    cnt_ref[...] += lax.dot_general(oh, ones_col, (((0,), (0,)), ((), ())),
                                    preferred_element_type=_f32)


def _run_k1(layer0, edge, dstf, psrc, feat, wmf_f, wcat, bias):
    ein = edge.shape[1]
    psrc_spec = (pl.BlockSpec((1, F), lambda i: (0, 0)) if layer0
                 else pl.BlockSpec((E_BLK, F), lambda i: (i, 1)))
    return pl.pallas_call(
        functools.partial(_k1_body, layer0),
        grid=(NB,),
        in_specs=[
            pl.BlockSpec((E_BLK, ein), lambda i: (i, 0)),
            pl.BlockSpec((E_BLK, 1), lambda i: (i, 0)),
            psrc_spec,
            pl.BlockSpec((F, F), lambda i: (0, 0)),
            pl.BlockSpec((F, F), lambda i: (0, 0)),
            pl.BlockSpec((ein, 3 * F), lambda i: (0, 0)),
            pl.BlockSpec((1, 3 * F), lambda i: (0, 0)),
        ],
        out_specs=[
            pl.BlockSpec((E_BLK, F), lambda i: (i, 0)),
            pl.BlockSpec((E_BLK, F), lambda i: (i, 0)),
            pl.BlockSpec((F, F), lambda i: (0, 0)),
            pl.BlockSpec((F, 1), lambda i: (0, 0)),
        ],
        out_shape=[
            jax.ShapeDtypeStruct((E, F), _f32),
            jax.ShapeDtypeStruct((E, F), _f32),
            jax.ShapeDtypeStruct((F, F), _f32),
            jax.ShapeDtypeStruct((F, 1), _f32),
        ],
        compiler_params=pltpu.CompilerParams(
            dimension_semantics=("arbitrary",)),
    )(edge, dstf, psrc, feat, wmf_f, wcat, bias)


def _k3_body(last, node_ref, aggo_ref, cntp_ref, aggf_ref, cntd_ref,
             feat_ref, wn_n_ref, wn_a_ref, bn_ref, wf_f_ref, wf_a_ref,
             bf_ref, we_n_ref, we_f_ref, wmo_nx_ref,
             node2_ref, *out_refs):
    if last:
        feat2_ref, = out_refs
    else:
        a_ref, feat2_ref, b_ref = out_refs
    node = node_ref[...]
    cnt = jnp.maximum(cntp_ref[0][:N_OBS, 0:1] + cntp_ref[1][:N_OBS, 0:1], 1.0)
    aggo = (aggo_ref[0][:N_OBS] + aggo_ref[1][:N_OBS]) / cnt
    new_node = jnp.maximum(
        jnp.dot(node, wn_n_ref[...], preferred_element_type=_f32)
        + jnp.dot(aggo, wn_a_ref[...], preferred_element_type=_f32)
        + bn_ref[...], 0.0)
    node2 = jnp.maximum(new_node + node, 0.0)
    node2_ref[...] = node2
    if not last:
        a = jnp.dot(new_node, we_n_ref[...], preferred_element_type=_f32)
        pn = jnp.dot(node2, wmo_nx_ref[...], preferred_element_type=_f32)
        a_ref[...] = jnp.concatenate([a, pn], axis=1)
    feat = feat_ref[...]
    aggf = aggf_ref[...] / jnp.maximum(cntd_ref[...], 1.0)
    new_feat = jnp.maximum(
        jnp.dot(feat, wf_f_ref[...], preferred_element_type=_f32)
        + jnp.dot(aggf, wf_a_ref[...], preferred_element_type=_f32)
        + bf_ref[...], 0.0)
    feat2 = jnp.maximum(new_feat + feat, 0.0)
    feat2_ref[...] = feat2
    if not last:
        b_ref[...] = jnp.dot(new_feat, we_f_ref[...],
                             preferred_element_type=_f32)


def _run_k3(last, node, aggo_parts, cnt_parts, aggf, cntd, feat,
            wn_n, wn_a, bn, wf_f, wf_a, bf, we_n, we_f, wmo_nx):
    return pl.pallas_call(
        functools.partial(_k3_body, last),
        out_shape=([jax.ShapeDtypeStruct((N_OBS, F), _f32),
                    jax.ShapeDtypeStruct((F, F), _f32)] if last else
                   [jax.ShapeDtypeStruct((N_OBS, F), _f32),
                    jax.ShapeDtypeStruct((N_OBS, 2 * F), _f32),
                    jax.ShapeDtypeStruct((F, F), _f32),
                    jax.ShapeDtypeStruct((F, F), _f32)]),
    )(node, aggo_parts, cnt_parts, aggf, cntd, feat,
      wn_n, wn_a, bn, wf_f, wf_a, bf, we_n, we_f, wmo_nx)


def _kb_body(mid, *refs):
    (epe_ref, g_ref, dstf_ref, prev_ref, b0_ref, be0_ref,
     feat_ref, wmf_f_ref, wcat_ref, bias_ref) = refs[:10]
    if mid:
        edge_ref, mof_ref, epe_out_ref, aggf_ref, cnt_ref = refs[10:]
    else:
        mof_ref, aggf_ref, cnt_ref = refs[10:]
    iota = lax.broadcasted_iota(jnp.int32, (1, F), 1).astype(_f32)
    oh = (dstf_ref[...] == iota).astype(_f32)
    g = g_ref[...]
    new_edge = jnp.maximum(
        epe_ref[...] + g[:, :F]
        + jnp.dot(oh, b0_ref[...], preferred_element_type=_f32)
        + be0_ref[...], 0.0)
    edge = jnp.maximum(new_edge + prev_ref[...], 0.0)
    if mid:
        edge_ref[...] = edge
    ep = jnp.dot(edge, wcat_ref[...], preferred_element_type=_f32)
    t = jnp.dot(feat_ref[...], wmf_f_ref[...], preferred_element_type=_f32)
    m_of = jnp.maximum(jnp.dot(oh, t, preferred_element_type=_f32)
                       + ep[:, :F] + bias_ref[:, :F], 0.0)
    mof_ref[...] = m_of
    m_fo = jnp.maximum(g[:, F:2 * F] + ep[:, F:2 * F]
                       + bias_ref[:, F:2 * F], 0.0)
    if mid:
        epe_out_ref[...] = ep[:, 2 * F:]

    @pl.when(pl.program_id(0) == 0)
    def _():
        aggf_ref[...] = jnp.zeros_like(aggf_ref)
        cnt_ref[...] = jnp.zeros_like(cnt_ref)

    aggf_ref[...] += lax.dot_general(oh, m_fo, (((0,), (0,)), ((), ())),
                                     preferred_element_type=_f32)
    ones_col = jnp.ones((E_BLK, 1), _f32)
    cnt_ref[...] += lax.dot_general(oh, ones_col, (((0,), (0,)), ((), ())),
                                    preferred_element_type=_f32)


def _run_kb(mid, epe, g, dstf, prev, b0, be0, feat, wmf_f, wcat, bias):
    ein = prev.shape[1]
    kw = wcat.shape[1]
    out_specs = [
        pl.BlockSpec((E_BLK, F), lambda i: (i, 0)),
        pl.BlockSpec((E_BLK, F), lambda i: (i, 0)),
        pl.BlockSpec((E_BLK, F), lambda i: (i, 0)),
        pl.BlockSpec((F, F), lambda i: (0, 0)),
        pl.BlockSpec((F, 1), lambda i: (0, 0)),
    ]
    out_shape = [
        jax.ShapeDtypeStruct((E, F), _f32),
        jax.ShapeDtypeStruct((E, F), _f32),
        jax.ShapeDtypeStruct((E, F), _f32),
        jax.ShapeDtypeStruct((F, F), _f32),
        jax.ShapeDtypeStruct((F, 1), _f32),
    ]
    if not mid:
        out_specs = out_specs[1:2] + out_specs[3:]
        out_shape = out_shape[1:2] + out_shape[3:]
    res = pl.pallas_call(
        functools.partial(_kb_body, mid),
        grid=(NB,),
        in_specs=[
            pl.BlockSpec((E_BLK, F), lambda i: (i, 0)),
            pl.BlockSpec((E_BLK, 2 * F), lambda i: (i, 0)),
            pl.BlockSpec((E_BLK, 1), lambda i: (i, 0)),
            pl.BlockSpec((E_BLK, ein), lambda i: (i, 0)),
            pl.BlockSpec((F, F), lambda i: (0, 0)),
            pl.BlockSpec((1, F), lambda i: (0, 0)),
            pl.BlockSpec((F, F), lambda i: (0, 0)),
            pl.BlockSpec((F, F), lambda i: (0, 0)),
            pl.BlockSpec((F, kw), lambda i: (0, 0)),
            pl.BlockSpec((1, kw), lambda i: (0, 0)),
        ],
        out_specs=out_specs,
        out_shape=out_shape,
        compiler_params=pltpu.CompilerParams(
            dimension_semantics=("arbitrary",)),
    )(epe, g, dstf, prev, b0, be0, feat, wmf_f, wcat, bias)
    if mid:
        edge, mof, epe_out, aggf, cnt = res
        return edge, mof, epe_out, aggf, cnt
    mof, aggf, cnt = res
    return None, mof, None, aggf, cnt


OB = 200


def _k6_body(node_ref, feat_ref, wo_ref, wfh_ref, bh_ref, wout_ref, bout_ref,
             w1_ref, b1_ref, w2_ref, b2_ref, dhat_ref, yhat_ref):
    obs_h = jnp.dot(node_ref[...], wo_ref[...], preferred_element_type=_f32)
    feat_h = jnp.dot(feat_ref[...], wfh_ref[...], preferred_element_type=_f32)
    h = jnp.maximum(obs_h[:, None, :] + feat_h[None, :, :]
                    + bh_ref[...][None, :, :], 0.0)
    dhat = jnp.sum(h * wout_ref[...][None, :, :], axis=2) + bout_ref[...]
    dhat_ref[...] = dhat
    hid = jnp.maximum(jnp.dot(dhat, w1_ref[...], preferred_element_type=_f32)
                      + b1_ref[...], 0.0)
    yhat_ref[...] = (jnp.dot(hid, w2_ref[...], preferred_element_type=_f32)
                     + b2_ref[...])


def _run_k6(node, feat, wo, wfh, bh, wout, bout, w1, b1, w2, b2):
    nblk = N_OBS // OB
    return pl.pallas_call(
        _k6_body,
        grid=(nblk,),
        in_specs=[
            pl.BlockSpec((OB, F), lambda i: (i, 0)),
            pl.BlockSpec((F, F), lambda i: (0, 0)),
            pl.BlockSpec((F, F), lambda i: (0, 0)),
            pl.BlockSpec((F, F), lambda i: (0, 0)),
            pl.BlockSpec((1, F), lambda i: (0, 0)),
            pl.BlockSpec((1, F), lambda i: (0, 0)),
            pl.BlockSpec((1, 1), lambda i: (0, 0)),
            pl.BlockSpec((F, F), lambda i: (0, 0)),
            pl.BlockSpec((1, F), lambda i: (0, 0)),
            pl.BlockSpec((F, 1), lambda i: (0, 0)),
            pl.BlockSpec((1, 1), lambda i: (0, 0)),
        ],
        out_specs=[
            pl.BlockSpec((OB, F), lambda i: (i, 0)),
            pl.BlockSpec((OB, 1), lambda i: (i, 0)),
        ],
        out_shape=[
            jax.ShapeDtypeStruct((N_OBS, F), _f32),
            jax.ShapeDtypeStruct((N_OBS, 1), _f32),
        ],
    )(node, feat, wo, wfh, bh, wout, bout, w1, b1, w2, b2)


@functools.lru_cache(maxsize=None)
def _sc_mesh():
    return plsc.VectorSubcoreMesh(core_axis_name="c", subcore_axis_name="s")


def _fill_vmem(ref, nrows, ncols, val):
    v = jnp.full((16,), val, _f32)

    def frow(r, _):
        def fcol(j, _):
            ref[r, pl.ds(j * 16, 16)] = v
            return 0
        return lax.fori_loop(0, ncols // 16, fcol, 0)
    lax.fori_loop(0, nrows, frow, 0)


RCH = 400
NRC = E_PER_W // RCH
SUB = RCH // CH
ZR = 80


def _scatter_body(with_count, mof_hbm, srcr_hbm, out_hbm, cnt_hbm,
                  rows_v, idx_v, zbuf_v, ones_v, cbuf_v, acc_sh, cacc_sh,
                  lsem, ssem, csem, zsem):
    cid = lax.axis_index("c")
    sid = lax.axis_index("s")
    wid = cid * NS + sid
    nrc = E_PER_W // RCH
    _fill_vmem(zbuf_v, ZR, F, 0.0)
    for p in range(RPAD // ZR):
        pltpu.async_copy(zbuf_v, acc_sh.at[pl.ds(sid * RPAD + p * ZR, ZR)],
                         zsem)
    if with_count:
        _fill_vmem(ones_v, CH, 16, 1.0)
        _fill_vmem(cbuf_v, ZR, 16, 0.0)
        for p in range(RPAD // ZR):
            pltpu.async_copy(cbuf_v,
                             cacc_sh.at[pl.ds(sid * RPAD + p * ZR, ZR)], zsem)
    for p in range(RPAD // ZR):
        pltpu.make_async_copy(
            zbuf_v, acc_sh.at[pl.ds(sid * RPAD + p * ZR, ZR)], zsem).wait()
        if with_count:
            pltpu.make_async_copy(
                cbuf_v, cacc_sh.at[pl.ds(sid * RPAD + p * ZR, ZR)],
                zsem).wait()
    pltpu.sync_copy(srcr_hbm.at[wid], idx_v)
    plsc.subcore_barrier()

    base = cid * E_PER_SC + sid * E_PER_W

    def mof_rows(co):
        return mof_hbm.at[pl.ds(base + co * RCH, RCH)]

    pltpu.async_copy(mof_rows(0), rows_v.at[0], lsem)

    def outer(co, _):
        b = co & 1
        pltpu.make_async_copy(mof_rows(co), rows_v.at[b], lsem).wait()

        @pl.when(co + 1 < nrc)
        def _():
            pltpu.async_copy(mof_rows(co + 1), rows_v.at[1 - b], lsem)
        for k in range(SUB):
            j = co * SUB + k
            pltpu.async_copy(rows_v.at[b, pl.ds(k * CH, CH)],
                             acc_sh.at[idx_v.at[j]], ssem, add=True)
            if with_count:
                pltpu.async_copy(ones_v, cacc_sh.at[idx_v.at[j]], csem,
                                 add=True)
        for k in range(SUB):
            pltpu.make_async_copy(rows_v.at[b, pl.ds(k * CH, CH)],
                                  acc_sh.at[idx_v.at[0]], ssem).wait()
            if with_count:
                pltpu.make_async_copy(ones_v, cacc_sh.at[idx_v.at[0]],
                                      csem).wait()
        return 0
    lax.fori_loop(0, nrc, outer, 0)
    plsc.subcore_barrier()
    for p in range(2):
        pltpu.sync_copy(acc_sh.at[pl.ds(sid * RPAD + p * 320, 320)],
                        rows_v.at[0, pl.ds(0, 320)])
        pltpu.sync_copy(rows_v.at[0, pl.ds(0, 320)],
                        out_hbm.at[pl.ds(cid * NPAD + sid * RPAD + p * 320,
                                         320)])
    if with_count:
        for p in range(RPAD // ZR):
            pltpu.sync_copy(cacc_sh.at[pl.ds(sid * RPAD + p * ZR, ZR)],
                            cbuf_v)
            pltpu.sync_copy(cbuf_v,
                            cnt_hbm.at[pl.ds(cid * NPAD + sid * RPAD + p * ZR,
                                             ZR)])


@functools.lru_cache(maxsize=None)
def _make_scatter(with_count):
    return pl.kernel(
        functools.partial(_scatter_body, with_count),
        out_type=[
            jax.ShapeDtypeStruct((NC * NPAD, F), _f32),
            jax.ShapeDtypeStruct((NC * NPAD, 16), _f32),
        ],
        mesh=_sc_mesh(),
        compiler_params=pltpu.CompilerParams(use_tc_tiling_on_sc=False),
        scratch_types=[
            pltpu.VMEM((2, RCH, F), _f32),
            pltpu.VMEM((NCH, CH), jnp.int32),
            pltpu.VMEM((ZR, F), _f32),
            pltpu.VMEM((CH, 16), _f32),
            pltpu.VMEM((ZR, 16), _f32),
            pltpu.VMEM_SHARED((NPAD, F), _f32),
            pltpu.VMEM_SHARED((NPAD, 16), _f32),
            pltpu.SemaphoreType.DMA,
            pltpu.SemaphoreType.DMA,
            pltpu.SemaphoreType.DMA,
            pltpu.SemaphoreType.DMA,
        ],
    )


def _sc_scatter(mof, srcr, with_count):
    out, cnt = _make_scatter(with_count)(mof, srcr)
    return out.reshape(NC, NPAD, F), cnt.reshape(NC, NPAD, 16)


def _gather_body(width, tab_hbm, srcr_hbm, out_hbm, idx_v, gb_v, gsem, wsem):
    cid = lax.axis_index("c")
    sid = lax.axis_index("s")
    wid = cid * NS + sid
    pltpu.sync_copy(srcr_hbm.at[wid], idx_v)
    base = cid * E_PER_SC + sid * E_PER_W

    pltpu.async_copy(tab_hbm.at[idx_v.at[0]], gb_v.at[0], gsem)

    def chunk(c, _):
        b = c & 1
        pltpu.make_async_copy(tab_hbm.at[idx_v.at[0]], gb_v.at[b],
                              gsem).wait()

        @pl.when(c >= 1)
        def _():
            pltpu.make_async_copy(gb_v.at[1 - b],
                                  out_hbm.at[pl.ds(base, CH)], wsem).wait()

        @pl.when(c + 1 < NCH)
        def _():
            pltpu.async_copy(tab_hbm.at[idx_v.at[c + 1]], gb_v.at[1 - b],
                             gsem)
        pltpu.async_copy(gb_v.at[b], out_hbm.at[pl.ds(base + c * CH, CH)],
                         wsem)
        return 0
    lax.fori_loop(0, NCH, chunk, 0)
    pltpu.make_async_copy(gb_v.at[0], out_hbm.at[pl.ds(base, CH)],
                          wsem).wait()


@functools.lru_cache(maxsize=None)
def _make_gather(width):
    return pl.kernel(
        functools.partial(_gather_body, width),
        out_type=jax.ShapeDtypeStruct((E, width), _f32),
        mesh=_sc_mesh(),
        compiler_params=pltpu.CompilerParams(use_tc_tiling_on_sc=False),
        scratch_types=[
            pltpu.VMEM((NCH, CH), jnp.int32),
            pltpu.VMEM((2, CH, width), _f32),
            pltpu.SemaphoreType.DMA,
            pltpu.SemaphoreType.DMA,
        ],
    )


def _sc_gather(tab, srcr):
    return _make_gather(tab.shape[1])(tab, srcr)


def kernel(x, edge_index, edge_value, params):
    src = edge_index[0].astype(jnp.int32)
    dstf = edge_index[1].astype(_f32).reshape(E, 1)
    n = N_OBS

    node_emb = jnp.ones((n, F), _f32)
    feature_emb = jnp.eye(F, dtype=_f32)
    edge_emb = edge_value.reshape(E, 1)

    def split(p, ein):
        wmf_f, wmf_e = p['Wmf'][:F], p['Wmf'][F:]
        wmo_n, wmo_e = p['Wmo'][:F], p['Wmo'][F:]
        we_e = p['We'][:ein]
        we_n = p['We'][ein:ein + F]
        we_f = p['We'][ein + F:]
        wcat = jnp.concatenate([wmf_e, wmo_e, we_e], axis=1)
        bias = jnp.concatenate([p['bmf'], p['bmo'], p['be']]).reshape(1, 3 * F)
        return wmf_f, wmo_n, wcat, bias, we_n, we_f

    blocks = [params['block%d' % i] for i in range(NUM_LAYERS)]
    sp = [split(blocks[i], 1 if i == 0 else F) for i in range(NUM_LAYERS)]

    srcr = src.reshape(NC * NS, NCH, CH)
    wmf_f0, wmo_n0, wcat0, bias0, we_n0, we_f0 = sp[0]
    wmf_f1, wmo_n1, wcat1, bias1, we_n1, we_f1 = sp[1]
    wmf_f2, wmo_n2, wcat2, bias2, _, _ = sp[2]

    psrc0 = jnp.sum(wmo_n0, axis=0, keepdims=True)
    mof0, epe0, aggf0, cntd = _run_k1(
        True, edge_emb, dstf, psrc0, feature_emb, wmf_f0, wcat0, bias0)
    aggo0, cnt_parts = _sc_scatter(mof0, srcr, with_count=True)
    node1, apn0, feat1, b0 = _run_k3(
        False, node_emb, aggo0, cnt_parts, aggf0, cntd, feature_emb,
        blocks[0]['Wn'][:F], blocks[0]['Wn'][F:], blocks[0]['bn'].reshape(1, F),
        blocks[0]['Wf'][:F], blocks[0]['Wf'][F:], blocks[0]['bf'].reshape(1, F),
        we_n0, we_f0, wmo_n1)
    g0 = _sc_gather(apn0, srcr)

    edge1, mof1, epe1, aggf1, _ = _run_kb(
        True, epe0, g0, dstf, edge_emb, b0, bias0[:, 2 * F:],
        feat1, wmf_f1, wcat1, bias1)
    aggo1, _ = _sc_scatter(mof1, srcr, with_count=False)
    node2, apn1, feat2, b1 = _run_k3(
        False, node1, aggo1, cnt_parts, aggf1, cntd, feat1,
        blocks[1]['Wn'][:F], blocks[1]['Wn'][F:], blocks[1]['bn'].reshape(1, F),
        blocks[1]['Wf'][:F], blocks[1]['Wf'][F:], blocks[1]['bf'].reshape(1, F),
        we_n1, we_f1, wmo_n2)
    g1 = _sc_gather(apn1, srcr)

    _, mof2, _, aggf2, _ = _run_kb(
        False, epe1, g1, dstf, edge1, b1, bias1[:, 2 * F:],
        feat2, wmf_f2, wcat2[:, :2 * F], bias2[:, :2 * F])
    aggo2, _ = _sc_scatter(mof2, srcr, with_count=False)
    node3, feat3 = _run_k3(
        True, node2, aggo2, cnt_parts, aggf2, cntd, feat2,
        blocks[2]['Wn'][:F], blocks[2]['Wn'][F:], blocks[2]['bn'].reshape(1, F),
        blocks[2]['Wf'][:F], blocks[2]['Wf'][F:], blocks[2]['bf'].reshape(1, F),
        we_n0, we_f0, wmo_n0)
    node_emb = node3
    feature_emb = feat3

    ep = params['eph']
    npar = params['nph']
    d_hat, y_hat = _run_k6(
        node_emb, feature_emb, ep['Wo'], ep['Wf'], ep['bh'].reshape(1, F),
        ep['wout'].reshape(1, F), ep['bout'].reshape(1, 1),
        npar['W1'], npar['b1'].reshape(1, F),
        npar['W2'], npar['b2'].reshape(1, 1))
    return d_hat, y_hat
